# baseline (speedup 1.0000x reference)
import functools

import jax
import jax.numpy as jnp
from jax import lax
from jax.experimental import pallas as pl
from jax.experimental.pallas import tpu as pltpu
from jax.experimental.pallas import tpu_sc as plsc

H = 26
D = 64
L = 16
OFF_TILE = 208


@functools.lru_cache(maxsize=None)
def _build(total):
    info = plsc.get_sparse_core_info()
    nc, ns = info.num_cores, info.num_subcores
    nw = nc * ns
    per_w = total // nw
    assert per_w * nw == total
    assert per_w % OFF_TILE == 0
    S = 512
    nch = per_w // S
    assert nch * S == per_w

    mesh = plsc.VectorSubcoreMesh(core_axis_name="c", subcore_axis_name="s")

    @functools.partial(
        pl.kernel,
        mesh=mesh,
        out_type=jax.ShapeDtypeStruct((total, D), jnp.float32),
        compiler_params=pltpu.CompilerParams(use_tc_tiling_on_sc=False),
        scratch_types=[
            pltpu.VMEM((per_w,), jnp.int32),
            pltpu.VMEM((OFF_TILE,), jnp.int32),
            pltpu.VMEM((2, S, D), jnp.float32),
            pltpu.SemaphoreType.DMA,
            pltpu.SemaphoreType.DMA,
            pltpu.SemaphoreType.DMA,
            pltpu.SemaphoreType.DMA,
        ],
    )
    def k(ids_hbm, table_hbm, off_hbm, out_hbm, idx_v, off_v, rows_v,
          g0, g1, w0, w1):
        wid = lax.axis_index("s") * nc + lax.axis_index("c")
        base = wid * per_w
        pltpu.sync_copy(off_hbm, off_v)
        pltpu.sync_copy(ids_hbm.at[pl.ds(base, per_w)], idx_v)

        def add_block(sb, carry):
            b0 = sb * OFF_TILE
            for k2 in range(OFF_TILE // L):
                sl = pl.ds(b0 + k2 * L, L)
                idx_v[sl] = idx_v[sl] + off_v[pl.ds(k2 * L, L)]
            return carry
        lax.fori_loop(0, per_w // OFF_TILE, add_block, 0)

        gsem = (g0, g1)
        wsem = (w0, w1)
        hg = [None, None]
        hw = [None, None]
        hg[0] = pltpu.async_copy(
            table_hbm.at[idx_v.at[pl.ds(0, S)]], rows_v.at[0], gsem[0])
        for c in range(nch):
            buf = c & 1
            nbuf = buf ^ 1
            if c + 1 < nch:
                if hw[nbuf] is not None:
                    hw[nbuf].wait()
                hg[nbuf] = pltpu.async_copy(
                    table_hbm.at[idx_v.at[pl.ds((c + 1) * S, S)]],
                    rows_v.at[nbuf], gsem[nbuf])
            hg[buf].wait()
            hw[buf] = pltpu.async_copy(
                rows_v.at[buf], out_hbm.at[pl.ds(base + c * S, S)], wsem[buf])
        hw[0].wait()
        hw[1].wait()

    return k


def kernel(input_ids, table, offsets):
    b, h = input_ids.shape
    ids_flat = input_ids.reshape(-1)
    off_tiled = jnp.tile(offsets, OFF_TILE // h)
    out = _build(b * h)(ids_flat, table, off_tiled)
    return out.reshape(b, h, D)


# speedup vs baseline: 1.0316x; 1.0316x over previous
import functools

import jax
import jax.numpy as jnp
from jax import lax
from jax.experimental import pallas as pl
from jax.experimental.pallas import tpu as pltpu
from jax.experimental.pallas import tpu_sc as plsc

H = 26
D = 64
DP = 128
L = 16
OFF_TILE = 208


@functools.lru_cache(maxsize=None)
def _build(total):
    info = plsc.get_sparse_core_info()
    nc, ns = info.num_cores, info.num_subcores
    nw = nc * ns
    per_w = total // nw
    assert per_w * nw == total
    assert per_w % OFF_TILE == 0
    S = 416
    nch = per_w // S
    assert nch * S == per_w

    mesh = plsc.VectorSubcoreMesh(core_axis_name="c", subcore_axis_name="s")

    @functools.partial(
        pl.kernel,
        mesh=mesh,
        out_type=jax.ShapeDtypeStruct((total, DP), jnp.float32),
        scratch_types=[
            pltpu.VMEM((per_w,), jnp.int32),
            pltpu.VMEM((OFF_TILE,), jnp.int32),
            pltpu.VMEM((2, S, DP), jnp.float32),
            pltpu.SemaphoreType.DMA,
            pltpu.SemaphoreType.DMA,
            pltpu.SemaphoreType.DMA,
            pltpu.SemaphoreType.DMA,
        ],
    )
    def k(ids_hbm, table_hbm, off_hbm, out_hbm, idx_v, off_v, rows_v,
          g0, g1, w0, w1):
        wid = lax.axis_index("s") * nc + lax.axis_index("c")
        base = wid * per_w
        pltpu.sync_copy(off_hbm, off_v)
        pltpu.sync_copy(ids_hbm.at[pl.ds(base, per_w)], idx_v)

        def add_block(sb, carry):
            b0 = sb * OFF_TILE
            for k2 in range(OFF_TILE // L):
                sl = pl.ds(b0 + k2 * L, L)
                idx_v[sl] = idx_v[sl] + off_v[pl.ds(k2 * L, L)]
            return carry
        lax.fori_loop(0, per_w // OFF_TILE, add_block, 0)

        gsem = (g0, g1)
        wsem = (w0, w1)
        hg = [None, None]
        hw = [None, None]
        hg[0] = pltpu.async_copy(
            table_hbm.at[idx_v.at[pl.ds(0, S)]], rows_v.at[0], gsem[0])
        for c in range(nch):
            buf = c & 1
            nbuf = buf ^ 1
            if c + 1 < nch:
                if hw[nbuf] is not None:
                    hw[nbuf].wait()
                hg[nbuf] = pltpu.async_copy(
                    table_hbm.at[idx_v.at[pl.ds((c + 1) * S, S)]],
                    rows_v.at[nbuf], gsem[nbuf])
            hg[buf].wait()
            hw[buf] = pltpu.async_copy(
                rows_v.at[buf], out_hbm.at[pl.ds(base + c * S, S)], wsem[buf])
        hw[0].wait()
        hw[1].wait()

    return k


def kernel(input_ids, table, offsets):
    b, h = input_ids.shape
    ids_flat = input_ids.reshape(-1)
    off_tiled = jnp.tile(offsets, OFF_TILE // h)
    table_p = jnp.pad(table, ((0, 0), (0, DP - D)))
    out = _build(b * h)(ids_flat, table_p, off_tiled)
    return out[:, :D].reshape(b, h, D)
